# speedup vs baseline: 1.2869x; 1.2869x over previous
import functools
import math

import jax
import jax.numpy as jnp
from jax import lax
from jax.experimental import pallas as pl
from jax.experimental.pallas import tpu as pltpu
from jax.experimental.pallas import tpu_sc as plsc

D_MODEL = 128
LANES = 16
NUM_CORES = 2
NUM_SUBCORES = 16
NUM_WORKERS = NUM_CORES * NUM_SUBCORES
CHUNK = 128
SCALE = math.sqrt(float(D_MODEL))


@functools.lru_cache(maxsize=None)
def _build(batch: int):
    assert batch % (NUM_WORKERS * CHUNK) == 0
    bpw = batch // NUM_WORKERS
    nchunk = bpw // CHUNK

    mesh = plsc.VectorSubcoreMesh(core_axis_name="c", subcore_axis_name="s")

    @functools.partial(
        pl.kernel,
        out_type=jax.ShapeDtypeStruct((batch, D_MODEL), jnp.float32),
        mesh=mesh,
        scratch_types=[
            pltpu.VMEM((nchunk, CHUNK), jnp.int32),
            pltpu.VMEM((bpw, D_MODEL), jnp.float32),
            pltpu.SemaphoreType.DMA,
        ],
    )
    def emb_kernel(x_hbm, lut_hbm, out_hbm, idx_v, rows_v, sem):
        wid = lax.axis_index("s") * NUM_CORES + lax.axis_index("c")
        base = wid * bpw

        for j in range(nchunk):
            pltpu.sync_copy(x_hbm.at[pl.ds(base + j * CHUNK, CHUNK)],
                            idx_v.at[j])

        copies = [
            pltpu.async_copy(lut_hbm.at[idx_v.at[j]],
                             rows_v.at[pl.ds(j * CHUNK, CHUNK)], sem)
            for j in range(nchunk)
        ]
        for c in copies:
            c.wait()

        @plsc.parallel_loop(0, bpw, unroll=4)
        def _(r):
            for c8 in range(D_MODEL // LANES):
                sl = rows_v[r, pl.ds(c8 * LANES, LANES)]
                rows_v[r, pl.ds(c8 * LANES, LANES)] = sl * SCALE

        pltpu.sync_copy(rows_v, out_hbm.at[pl.ds(base, bpw)])

    return emb_kernel


def kernel(x, lut):
    b0, b1 = x.shape
    xf = x.reshape(-1).astype(jnp.int32)
    out = _build(b0 * b1)(xf, lut)
    return out.reshape(b0, b1, D_MODEL)


# speedup vs baseline: 1.4043x; 1.0912x over previous
import functools
import math

import jax
import jax.numpy as jnp
from jax import lax
from jax.experimental import pallas as pl
from jax.experimental.pallas import tpu as pltpu
from jax.experimental.pallas import tpu_sc as plsc

D_MODEL = 128
LANES = 16
NUM_CORES = 2
NUM_SUBCORES = 16
NUM_WORKERS = NUM_CORES * NUM_SUBCORES
CHUNK = 128
SCALE = math.sqrt(float(D_MODEL))


@functools.lru_cache(maxsize=None)
def _build(batch: int):
    assert batch % (NUM_WORKERS * CHUNK) == 0
    bpw = batch // NUM_WORKERS
    nchunk = bpw // CHUNK

    mesh = plsc.VectorSubcoreMesh(core_axis_name="c", subcore_axis_name="s")

    @functools.partial(
        pl.kernel,
        out_type=jax.ShapeDtypeStruct((batch, D_MODEL), jnp.float32),
        mesh=mesh,
        scratch_types=[
            pltpu.VMEM((nchunk, CHUNK), jnp.int32),
            pltpu.VMEM((bpw, D_MODEL), jnp.float32),
            [pltpu.SemaphoreType.DMA] * nchunk,
            pltpu.SemaphoreType.DMA,
        ],
    )
    def emb_kernel(x_hbm, lut_hbm, out_hbm, idx_v, rows_v, gsems, wsem):
        wid = lax.axis_index("s") * NUM_CORES + lax.axis_index("c")
        base = wid * bpw

        pltpu.sync_copy(x_hbm.at[wid], idx_v)

        gathers = [
            pltpu.async_copy(lut_hbm.at[idx_v.at[j]],
                             rows_v.at[pl.ds(j * CHUNK, CHUNK)], gsems[j])
            for j in range(nchunk)
        ]

        writes = []
        for j in range(nchunk):
            gathers[j].wait()

            @plsc.parallel_loop(j * CHUNK, (j + 1) * CHUNK, unroll=4)
            def _(r):
                for c8 in range(D_MODEL // LANES):
                    sl = rows_v[r, pl.ds(c8 * LANES, LANES)]
                    rows_v[r, pl.ds(c8 * LANES, LANES)] = sl * SCALE

            writes.append(
                pltpu.async_copy(rows_v.at[pl.ds(j * CHUNK, CHUNK)],
                                 out_hbm.at[pl.ds(base + j * CHUNK, CHUNK)],
                                 wsem))
        for w in writes:
            w.wait()

    return emb_kernel


def kernel(x, lut):
    b0, b1 = x.shape
    batch = b0 * b1
    xf = x.reshape(NUM_WORKERS, -1, CHUNK).astype(jnp.int32)
    out = _build(batch)(xf, lut)
    return out.reshape(b0, b1, D_MODEL)


# speedup vs baseline: 1.4088x; 1.0032x over previous
import functools
import math

import jax
import jax.numpy as jnp
from jax import lax
from jax.experimental import pallas as pl
from jax.experimental.pallas import tpu as pltpu
from jax.experimental.pallas import tpu_sc as plsc

D_MODEL = 128
LANES = 16
NUM_CORES = 2
NUM_SUBCORES = 16
NUM_WORKERS = NUM_CORES * NUM_SUBCORES
CHUNK = 128
SCALE = math.sqrt(float(D_MODEL))


@functools.lru_cache(maxsize=None)
def _build(batch: int):
    assert batch % (NUM_WORKERS * CHUNK) == 0
    bpw = batch // NUM_WORKERS
    nchunk = bpw // CHUNK

    mesh = plsc.VectorSubcoreMesh(core_axis_name="c", subcore_axis_name="s")

    @functools.partial(
        pl.kernel,
        out_type=jax.ShapeDtypeStruct((batch, D_MODEL), jnp.float32),
        mesh=mesh,
        scratch_types=[
            pltpu.VMEM((nchunk, CHUNK), jnp.int32),
            pltpu.VMEM((bpw, D_MODEL), jnp.float32),
            [pltpu.SemaphoreType.DMA] * nchunk,
            [pltpu.SemaphoreType.DMA] * nchunk,
            pltpu.SemaphoreType.DMA,
        ],
    )
    def emb_kernel(x_hbm, lut_hbm, out_hbm, idx_v, rows_v, isems, gsems, wsem):
        wid = lax.axis_index("s") * NUM_CORES + lax.axis_index("c")
        base = wid * bpw

        iloads = [
            pltpu.async_copy(x_hbm.at[pl.ds(base + j * CHUNK, CHUNK)],
                             idx_v.at[j], isems[j])
            for j in range(nchunk)
        ]

        gathers = []
        for j in range(nchunk):
            iloads[j].wait()
            gathers.append(
                pltpu.async_copy(lut_hbm.at[idx_v.at[j]],
                                 rows_v.at[pl.ds(j * CHUNK, CHUNK)], gsems[j]))

        writes = []
        for j in range(nchunk):
            gathers[j].wait()

            @plsc.parallel_loop(j * CHUNK, (j + 1) * CHUNK, unroll=4)
            def _(r):
                for c8 in range(D_MODEL // LANES):
                    sl = rows_v[r, pl.ds(c8 * LANES, LANES)]
                    rows_v[r, pl.ds(c8 * LANES, LANES)] = sl * SCALE

            writes.append(
                pltpu.async_copy(rows_v.at[pl.ds(j * CHUNK, CHUNK)],
                                 out_hbm.at[pl.ds(base + j * CHUNK, CHUNK)],
                                 wsem))
        for w in writes:
            w.wait()

    return emb_kernel


def kernel(x, lut):
    b0, b1 = x.shape
    batch = b0 * b1
    xf = jnp.ravel(x)
    if xf.dtype != jnp.int32:
        xf = xf.astype(jnp.int32)
    out = _build(batch)(xf, lut)
    return out.reshape(b0, b1, D_MODEL)
